# baseline (speedup 1.0000x reference)
import functools

import jax
import jax.numpy as jnp
from jax import lax
from jax.experimental import pallas as pl
from jax.experimental.pallas import tpu as pltpu
from jax.experimental.pallas import tpu_sc as plsc

B = 1024
L = 512
D = 768
DW = D // 2
N_TOK = B * L
EPS = 1e-5

_NC = 2
_NS = 16
_NW = _NC * _NS
_SEG = 8
_SEG_ROWS = N_TOK // _SEG
_RPW = _SEG_ROWS // _NW
_CHUNK = 128
_NCHUNK = _RPW // _CHUNK


def _sc_gather_body(ids_hbm, table_hbm, out_hbm, idx_v, rows0, rows1,
                    gsem0, gsem1):
    wid = lax.axis_index("s") * _NC + lax.axis_index("c")
    base = wid * _RPW
    cbase = wid * _NCHUNK

    def out_at(i):
        return out_hbm.at[pl.ds(base + i * _CHUNK, _CHUNK)]

    pltpu.sync_copy(ids_hbm.at[pl.ds(cbase, _NCHUNK)], idx_v)

    def gather(buf, idx_row, sem):
        return pltpu.async_copy(table_hbm.at[idx_v.at[idx_row]], buf, sem)

    gather(rows0, 0, gsem0)

    def pair(j, carry):
        i0 = 2 * j
        pltpu.make_async_copy(table_hbm.at[idx_v.at[i0]], rows0, gsem0).wait()
        gather(rows1, i0 + 1, gsem1)
        pltpu.sync_copy(rows0, out_at(i0))
        pltpu.make_async_copy(table_hbm.at[idx_v.at[i0 + 1]], rows1,
                              gsem1).wait()
        gather(rows0, i0 + 2, gsem0)
        pltpu.sync_copy(rows1, out_at(i0 + 1))
        return carry

    lax.fori_loop(0, _NCHUNK // 2 - 1, pair, 0)

    i0 = _NCHUNK - 2
    pltpu.make_async_copy(table_hbm.at[idx_v.at[i0]], rows0, gsem0).wait()
    gather(rows1, i0 + 1, gsem1)
    pltpu.sync_copy(rows0, out_at(i0))
    pltpu.make_async_copy(table_hbm.at[idx_v.at[i0 + 1]], rows1, gsem1).wait()
    pltpu.sync_copy(rows1, out_at(i0 + 1))


_sc_gather = functools.partial(
    pl.kernel,
    mesh=plsc.VectorSubcoreMesh(core_axis_name="c", subcore_axis_name="s"),
    out_type=jax.ShapeDtypeStruct((_SEG_ROWS, DW), jnp.int32),
    scratch_types=[
        pltpu.VMEM((_NCHUNK, _CHUNK), jnp.int32),
        pltpu.VMEM((_CHUNK, DW), jnp.int32),
        pltpu.VMEM((_CHUNK, DW), jnp.int32),
        pltpu.SemaphoreType.DMA,
        pltpu.SemaphoreType.DMA,
    ],
)(_sc_gather_body)


def _ln_first_body(x_ref, add_ref, gamma_ref, beta_ref, o_ref):
    x = x_ref[...].astype(jnp.float32) + add_ref[...]
    m = jnp.mean(x, axis=-1, keepdims=True)
    d = x - m
    v = jnp.mean(d * d, axis=-1, keepdims=True)
    o_ref[...] = d * lax.rsqrt(v + EPS) * gamma_ref[...] + beta_ref[...]


def _ln_body(x_ref, add_ref, gamma_ref, beta_ref, acc_ref, o_ref):
    del acc_ref
    _ln_first_body(x_ref, add_ref, gamma_ref, beta_ref, o_ref)


def _ln_seg(seg, buf, addvec, gamma2d, beta2d, acc):
    base_blk = seg * (_SEG_ROWS // L)
    return pl.pallas_call(
        _ln_body,
        grid=(_SEG_ROWS // L,),
        in_specs=[
            pl.BlockSpec((L, D), lambda i: (i, 0)),
            pl.BlockSpec((L, D), lambda i: (0, 0)),
            pl.BlockSpec((1, D), lambda i: (0, 0)),
            pl.BlockSpec((1, D), lambda i: (0, 0)),
            pl.BlockSpec(memory_space=pl.ANY),
        ],
        out_specs=pl.BlockSpec((L, D), lambda i: (base_blk + i, 0)),
        out_shape=jax.ShapeDtypeStruct((N_TOK, D), jnp.float32),
        input_output_aliases={4: 0},
    )(buf, addvec, gamma2d, beta2d, acc)


def _ln_first(buf, addvec, gamma2d, beta2d):
    return pl.pallas_call(
        _ln_first_body,
        grid=(_SEG_ROWS // L,),
        in_specs=[
            pl.BlockSpec((L, D), lambda i: (i, 0)),
            pl.BlockSpec((L, D), lambda i: (0, 0)),
            pl.BlockSpec((1, D), lambda i: (0, 0)),
            pl.BlockSpec((1, D), lambda i: (0, 0)),
        ],
        out_specs=pl.BlockSpec((L, D), lambda i: (i, 0)),
        out_shape=jax.ShapeDtypeStruct((N_TOK, D), jnp.float32),
    )(buf, addvec, gamma2d, beta2d)


def kernel(input_ids, modality_type, table, pos_emb, mod_emb, gamma, beta):
    ids = input_ids.reshape(N_TOK // _CHUNK, _CHUNK).astype(jnp.int32)
    mod_row = lax.dynamic_index_in_dim(mod_emb, modality_type, axis=0,
                                       keepdims=False)
    addvec = pos_emb[0, :L, :] + mod_row[None, :]
    gamma2d = gamma.reshape(1, D)
    beta2d = beta.reshape(1, D)

    tbl_i32 = lax.bitcast_convert_type(
        table.astype(jnp.bfloat16).reshape(table.shape[0], DW, 2),
        jnp.int32)

    seg_id_rows = _SEG_ROWS // _CHUNK
    bufs = [_sc_gather(lax.dynamic_slice_in_dim(ids, s * seg_id_rows,
                                                seg_id_rows), tbl_i32)
            for s in range(_SEG)]

    def as_bf16(buf):
        return lax.bitcast_convert_type(buf, jnp.bfloat16).reshape(
            _SEG_ROWS, D)

    acc = _ln_first(as_bf16(bufs[0]), addvec, gamma2d, beta2d)
    for s in range(1, _SEG):
        acc = _ln_seg(s, as_bf16(bufs[s]), addvec, gamma2d, beta2d, acc)
    return acc.reshape(B, L, D)


# speedup vs baseline: 5.2232x; 5.2232x over previous
import functools

import jax
import jax.numpy as jnp
from jax import lax
from jax.experimental import pallas as pl
from jax.experimental.pallas import tpu as pltpu
from jax.experimental.pallas import tpu_sc as plsc

B = 1024
L = 512
D = 768
DW = D // 2
N_TOK = B * L
EPS = 1e-5

_NC = 2
_NS = 16
_NW = _NC * _NS
_SEG = 8
_SEG_ROWS = N_TOK // _SEG
_RPW = _SEG_ROWS // _NW
_CHUNK = 128
_NCHUNK = _RPW // _CHUNK


def _sc_gather_body(ids_hbm, table_hbm, out_hbm, idx_v, rows0, rows1,
                    gsem0, gsem1):
    wid = lax.axis_index("s") * _NC + lax.axis_index("c")
    base = wid * _RPW
    cbase = wid * _NCHUNK

    def out_at(i):
        return out_hbm.at[pl.ds(base + i * _CHUNK, _CHUNK)]

    pltpu.sync_copy(ids_hbm.at[pl.ds(cbase, _NCHUNK)], idx_v)

    def gather(buf, idx_row, sem):
        return pltpu.async_copy(table_hbm.at[idx_v.at[idx_row]], buf, sem)

    gather(rows0, 0, gsem0)

    def pair(j, carry):
        i0 = 2 * j
        pltpu.make_async_copy(table_hbm.at[idx_v.at[i0]], rows0, gsem0).wait()
        gather(rows1, i0 + 1, gsem1)
        pltpu.sync_copy(rows0, out_at(i0))
        pltpu.make_async_copy(table_hbm.at[idx_v.at[i0 + 1]], rows1,
                              gsem1).wait()
        gather(rows0, i0 + 2, gsem0)
        pltpu.sync_copy(rows1, out_at(i0 + 1))
        return carry

    lax.fori_loop(0, _NCHUNK // 2 - 1, pair, 0)

    i0 = _NCHUNK - 2
    pltpu.make_async_copy(table_hbm.at[idx_v.at[i0]], rows0, gsem0).wait()
    gather(rows1, i0 + 1, gsem1)
    pltpu.sync_copy(rows0, out_at(i0))
    pltpu.make_async_copy(table_hbm.at[idx_v.at[i0 + 1]], rows1, gsem1).wait()
    pltpu.sync_copy(rows1, out_at(i0 + 1))


_sc_gather = functools.partial(
    pl.kernel,
    mesh=plsc.VectorSubcoreMesh(core_axis_name="c", subcore_axis_name="s"),
    out_type=jax.ShapeDtypeStruct((_SEG_ROWS, DW), jnp.int32),
    scratch_types=[
        pltpu.VMEM((_NCHUNK, _CHUNK), jnp.int32),
        pltpu.VMEM((_CHUNK, DW), jnp.int32),
        pltpu.VMEM((_CHUNK, DW), jnp.int32),
        pltpu.SemaphoreType.DMA,
        pltpu.SemaphoreType.DMA,
    ],
)(_sc_gather_body)


def _ln_first_body(x_ref, add_ref, gamma_ref, beta_ref, o_ref):
    xi = x_ref[...]
    f_lo = lax.bitcast_convert_type(xi << 16, jnp.float32)
    f_hi = lax.bitcast_convert_type(xi & jnp.int32(-65536), jnp.float32)
    x = jnp.concatenate([f_lo, f_hi], axis=-1) + add_ref[...]
    m = jnp.mean(x, axis=-1, keepdims=True)
    d = x - m
    v = jnp.mean(d * d, axis=-1, keepdims=True)
    o_ref[...] = d * lax.rsqrt(v + EPS) * gamma_ref[...] + beta_ref[...]


def _ln_body(x_ref, add_ref, gamma_ref, beta_ref, acc_ref, o_ref):
    del acc_ref
    _ln_first_body(x_ref, add_ref, gamma_ref, beta_ref, o_ref)


def _ln_seg(seg, buf, addvec, gamma2d, beta2d, acc):
    base_blk = seg * (_SEG_ROWS // L)
    return pl.pallas_call(
        _ln_body,
        grid=(_SEG_ROWS // L,),
        in_specs=[
            pl.BlockSpec((L, DW), lambda i: (i, 0)),
            pl.BlockSpec((L, D), lambda i: (0, 0)),
            pl.BlockSpec((1, D), lambda i: (0, 0)),
            pl.BlockSpec((1, D), lambda i: (0, 0)),
            pl.BlockSpec(memory_space=pl.ANY),
        ],
        out_specs=pl.BlockSpec((L, D), lambda i: (base_blk + i, 0)),
        out_shape=jax.ShapeDtypeStruct((N_TOK, D), jnp.float32),
        input_output_aliases={4: 0},
    )(buf, addvec, gamma2d, beta2d, acc)


def _ln_first(buf, addvec, gamma2d, beta2d):
    return pl.pallas_call(
        _ln_first_body,
        grid=(_SEG_ROWS // L,),
        in_specs=[
            pl.BlockSpec((L, DW), lambda i: (i, 0)),
            pl.BlockSpec((L, D), lambda i: (0, 0)),
            pl.BlockSpec((1, D), lambda i: (0, 0)),
            pl.BlockSpec((1, D), lambda i: (0, 0)),
        ],
        out_specs=pl.BlockSpec((L, D), lambda i: (i, 0)),
        out_shape=jax.ShapeDtypeStruct((N_TOK, D), jnp.float32),
    )(buf, addvec, gamma2d, beta2d)


def kernel(input_ids, modality_type, table, pos_emb, mod_emb, gamma, beta):
    ids = input_ids.reshape(N_TOK // _CHUNK, _CHUNK).astype(jnp.int32)
    mod_row = lax.dynamic_index_in_dim(mod_emb, modality_type, axis=0,
                                       keepdims=False)
    addvec = pos_emb[0, :L, :] + mod_row[None, :]
    gamma2d = gamma.reshape(1, D)
    beta2d = beta.reshape(1, D)

    tb = lax.bitcast_convert_type(table.astype(jnp.bfloat16),
                                  jnp.uint16).astype(jnp.uint32)
    tbl_i32 = (tb[:, :DW] | (tb[:, DW:] << 16)).astype(jnp.int32)

    seg_id_rows = _SEG_ROWS // _CHUNK
    bufs = [_sc_gather(lax.dynamic_slice_in_dim(ids, s * seg_id_rows,
                                                seg_id_rows), tbl_i32)
            for s in range(_SEG)]

    acc = _ln_first(bufs[0], addvec, gamma2d, beta2d)
    for s in range(1, _SEG):
        acc = _ln_seg(s, bufs[s], addvec, gamma2d, beta2d, acc)
    return acc.reshape(B, L, D)
